# baseline (speedup 1.0000x reference)
import functools

import jax
import jax.numpy as jnp
from jax.experimental import pallas as pl
from jax.experimental.pallas import tpu as pltpu

R = 2048
C = 128
N = R * C
LOGN = 18
B = 8


def _loss_kernel(logits_ref, labels_ref, out_ref):
    lg = logits_ref[...].reshape(R, C)
    lb = labels_ref[...].reshape(R, C)
    lbf = lb.astype(jnp.float32)
    signs = 2.0 * lbf - 1.0
    e = 1.0 - lg * signs

    maxe = jnp.max(e)

    bits = jax.lax.bitcast_convert_type(e, jnp.int32)
    ikey = bits ^ ((bits >> 31) & jnp.int32(0x7FFFFFFF))
    x = (ikey & jnp.int32(~1)) | lb

    idx = (jax.lax.broadcasted_iota(jnp.int32, (R, C), 1) * R
           + jax.lax.broadcasted_iota(jnp.int32, (R, C), 0))

    w = x ^ (-((idx >> 1) & 1))
    for kk in range(1, LOGN + 1):
        for j in range(kk - 1, -1, -1):
            s = 1 << j
            if s >= R:
                t = s // R
                liota = jax.lax.broadcasted_iota(jnp.int32, (1, C), 1)
                is_lower = (liota & t) == 0
                down = pltpu.roll(w, C - t, axis=1)
                hi = jnp.maximum(w, down)
                lo = jnp.minimum(w, down)
                w = jnp.where(is_lower, hi, pltpu.roll(lo, t, axis=1))
            elif s >= 8:
                O = R // (2 * s)
                y = w.reshape(O, 2, s, C)
                a = y[:, 0, :, :]
                b = y[:, 1, :, :]
                w = jnp.concatenate(
                    [jnp.maximum(a, b)[:, None], jnp.minimum(a, b)[:, None]],
                    axis=1).reshape(R, C)
            else:
                y = w.reshape(R // 8, 8, C)
                io8 = jax.lax.broadcasted_iota(jnp.int32, (1, 8, C), 1)
                is_lower = (io8 & s) == 0
                down = pltpu.roll(y, 8 - s, axis=1)
                hi = jnp.maximum(y, down)
                lo = jnp.minimum(y, down)
                w = jnp.where(is_lower, hi, pltpu.roll(lo, s, axis=1)).reshape(R, C)
        if kk < LOGN:
            w = w ^ (-(((idx >> kk) ^ (idx >> (kk + 1))) & 1))
    x = w

    gt = (x & 1).astype(jnp.float32)
    bdec = jnp.where(x >= 0, x, x ^ jnp.int32(0x7FFFFFFF))
    e_s = jax.lax.bitcast_convert_type(bdec, jnp.float32)
    relu_e = jnp.maximum(e_s, 0.0)

    g8 = gt.reshape(R // 8, 8, C)
    io8 = jax.lax.broadcasted_iota(jnp.int32, (1, 8, C), 1)
    d = 1
    while d < 8:
        shifted = pltpu.roll(g8, d, axis=1)
        g8 = g8 + jnp.where(io8 >= d, shifted, 0.0)
        d *= 2
    osum = g8[:, 7, :]
    oiota = jax.lax.broadcasted_iota(jnp.int32, (R // 8, C), 0)
    d = 1
    while d < R // 8:
        shifted = pltpu.roll(osum, d, axis=0)
        osum = osum + jnp.where(oiota >= d, shifted, 0.0)
        d *= 2
    oex = osum - g8[:, 7, :]
    col = (g8 + oex[:, None, :]).reshape(R, C)
    colsum = jax.lax.slice(col, (R - 1, 0), (R, C))
    liota = jax.lax.broadcasted_iota(jnp.int32, (1, C), 1)
    lane = colsum
    d = 1
    while d < C:
        shifted = pltpu.roll(lane, d, axis=1)
        lane = lane + jnp.where(liota >= d, shifted, 0.0)
        d *= 2
    G = jax.lax.slice(lane, (0, C - 1), (1, C))[0, 0]
    lane_ex = lane - colsum
    c = col + lane_ex

    kpos = idx.astype(jnp.float32)
    U = G + (kpos + 1.0) - c
    Um1 = jnp.maximum(U - 1.0 + gt, 1.0)
    grad = gt / U + (1.0 - gt) * (G - c) / (U * Um1)
    loss = jnp.sum(relu_e * grad)
    loss = jnp.where(G > 0.0, loss, jnp.maximum(maxe, 0.0))
    out_ref[...] = jnp.full((1, 1, C), loss, dtype=jnp.float32)


@jax.jit
def kernel(logits, labels):
    lg = logits.reshape(B, R, C)
    lb = labels.reshape(B, R, C)
    out = pl.pallas_call(
        _loss_kernel,
        grid=(B,),
        in_specs=[
            pl.BlockSpec((1, R, C), lambda i: (i, 0, 0)),
            pl.BlockSpec((1, R, C), lambda i: (i, 0, 0)),
        ],
        out_specs=pl.BlockSpec((1, 1, C), lambda i: (i, 0, 0)),
        out_shape=jax.ShapeDtypeStruct((B, 1, C), jnp.float32),
    )(lg, lb)
    return jnp.mean(out[:, 0, 0])


# speedup vs baseline: 1.0197x; 1.0197x over previous
import functools

import jax
import jax.numpy as jnp
from jax.experimental import pallas as pl
from jax.experimental.pallas import tpu as pltpu

R = 2048
C = 128
N = R * C
LOGN = 18
B = 8


def _loss_kernel(logits_ref, labels_ref, out_ref):
    lg = logits_ref[...].reshape(R, C)
    lb = labels_ref[...].reshape(R, C)
    lbf = lb.astype(jnp.float32)
    signs = 2.0 * lbf - 1.0
    e = 1.0 - lg * signs

    maxe = jnp.max(e)

    bits = jax.lax.bitcast_convert_type(e, jnp.int32)
    ikey = bits ^ ((bits >> 31) & jnp.int32(0x7FFFFFFF))
    x = (ikey & jnp.int32(~1)) | lb

    idx = (jax.lax.broadcasted_iota(jnp.int32, (R, C), 1) * R
           + jax.lax.broadcasted_iota(jnp.int32, (R, C), 0))

    gray = idx ^ (idx >> 1)
    w = x ^ (-((idx >> 1) & 1))
    for kk in range(1, LOGN + 1):
        for j in range(kk - 1, -1, -1):
            s = 1 << j
            if s >= R:
                t = s // R
                liota = jax.lax.broadcasted_iota(jnp.int32, (1, C), 1)
                is_lower = (liota & t) == 0
                down = pltpu.roll(w, C - t, axis=1)
                hi = jnp.maximum(w, down)
                lo = jnp.minimum(w, down)
                w = jnp.where(is_lower, hi, pltpu.roll(lo, t, axis=1))
            elif s >= 8:
                O = R // (2 * s)
                y = w.reshape(O, 2, s, C)
                a = y[:, 0, :, :]
                b = y[:, 1, :, :]
                w = jnp.concatenate(
                    [jnp.maximum(a, b)[:, None], jnp.minimum(a, b)[:, None]],
                    axis=1).reshape(R, C)
            else:
                y = w.reshape(R // 8, 8, C)
                io8 = jax.lax.broadcasted_iota(jnp.int32, (1, 8, C), 1)
                is_lower = (io8 & s) == 0
                down = pltpu.roll(y, 8 - s, axis=1)
                hi = jnp.maximum(y, down)
                lo = jnp.minimum(y, down)
                w = jnp.where(is_lower, hi, pltpu.roll(lo, s, axis=1)).reshape(R, C)
        if kk < LOGN:
            w = w ^ ((gray << (31 - kk)) >> 31)
    x = w

    gt = (x & 1).astype(jnp.float32)
    bdec = jnp.where(x >= 0, x, x ^ jnp.int32(0x7FFFFFFF))
    e_s = jax.lax.bitcast_convert_type(bdec, jnp.float32)
    relu_e = jnp.maximum(e_s, 0.0)

    col = gt
    riota = jax.lax.broadcasted_iota(jnp.int32, (R, C), 0)
    d = 1
    while d < R:
        shifted = pltpu.roll(col, d, axis=0)
        col = col + jnp.where(riota >= d, shifted, 0.0)
        d *= 2
    colsum = jax.lax.slice(col, (R - 1, 0), (R, C))
    liota = jax.lax.broadcasted_iota(jnp.int32, (1, C), 1)
    lane = colsum
    d = 1
    while d < C:
        shifted = pltpu.roll(lane, d, axis=1)
        lane = lane + jnp.where(liota >= d, shifted, 0.0)
        d *= 2
    G = jax.lax.slice(lane, (0, C - 1), (1, C))[0, 0]
    lane_ex = lane - colsum
    c = col + lane_ex

    kpos = idx.astype(jnp.float32)
    U = G + (kpos + 1.0) - c
    Um1 = jnp.maximum(U - 1.0 + gt, 1.0)
    grad = gt / U + (1.0 - gt) * (G - c) / (U * Um1)
    loss = jnp.sum(relu_e * grad)
    loss = jnp.where(G > 0.0, loss, jnp.maximum(maxe, 0.0))
    out_ref[...] = jnp.full((1, 1, C), loss, dtype=jnp.float32)


@jax.jit
def kernel(logits, labels):
    lg = logits.reshape(B, R, C)
    lb = labels.reshape(B, R, C)
    out = pl.pallas_call(
        _loss_kernel,
        grid=(B,),
        in_specs=[
            pl.BlockSpec((1, R, C), lambda i: (i, 0, 0)),
            pl.BlockSpec((1, R, C), lambda i: (i, 0, 0)),
        ],
        out_specs=pl.BlockSpec((1, 1, C), lambda i: (i, 0, 0)),
        out_shape=jax.ShapeDtypeStruct((B, 1, C), jnp.float32),
    )(lg, lb)
    return jnp.mean(out[:, 0, 0])


# speedup vs baseline: 1.1456x; 1.1235x over previous
import jax
import jax.numpy as jnp
from jax.experimental import pallas as pl
from jax.experimental.pallas import tpu as pltpu

R = 2048
C = 128
N = R * C
LOGN = 18
B = 8
IB = 2


def _loss_kernel(logits_ref, labels_ref, out_ref):
    lg = logits_ref[...]
    lb = labels_ref[...]
    lbf = lb.astype(jnp.float32)
    signs = 2.0 * lbf - 1.0
    e = 1.0 - lg * signs

    maxe = jnp.max(e, axis=(1, 2), keepdims=True)

    bits = jax.lax.bitcast_convert_type(e, jnp.int32)
    ikey = bits ^ ((bits >> 31) & jnp.int32(0x7FFFFFFF))
    x = (ikey & jnp.int32(~1)) | lb

    idx = (jax.lax.broadcasted_iota(jnp.int32, (R, C), 1) * R
           + jax.lax.broadcasted_iota(jnp.int32, (R, C), 0))

    gray = idx ^ (idx >> 1)
    w = x ^ (-((idx >> 1) & 1))
    for kk in range(1, LOGN + 1):
        for j in range(kk - 1, -1, -1):
            s = 1 << j
            if s >= R:
                t = s // R
                liota = jax.lax.broadcasted_iota(jnp.int32, (1, C), 1)
                is_lower = (liota & t) == 0
                down = pltpu.roll(w, C - t, axis=2)
                hi = jnp.maximum(w, down)
                lo = jnp.minimum(w, down)
                w = jnp.where(is_lower, hi, pltpu.roll(lo, t, axis=2))
            elif s >= 8:
                O = R // (2 * s)
                y = w.reshape(IB, O, 2, s, C)
                a = y[:, :, 0, :, :]
                b = y[:, :, 1, :, :]
                w = jnp.concatenate(
                    [jnp.maximum(a, b)[:, :, None], jnp.minimum(a, b)[:, :, None]],
                    axis=2).reshape(IB, R, C)
            else:
                y = w.reshape(IB, R // 8, 8, C)
                io8 = jax.lax.broadcasted_iota(jnp.int32, (1, 8, C), 1)
                is_lower = (io8 & s) == 0
                down = pltpu.roll(y, 8 - s, axis=2)
                hi = jnp.maximum(y, down)
                lo = jnp.minimum(y, down)
                w = jnp.where(is_lower, hi,
                              pltpu.roll(lo, s, axis=2)).reshape(IB, R, C)
        if kk < LOGN:
            w = w ^ ((gray << (31 - kk)) >> 31)
    x = w

    gt = (x & 1).astype(jnp.float32)
    bdec = jnp.where(x >= 0, x, x ^ jnp.int32(0x7FFFFFFF))
    e_s = jax.lax.bitcast_convert_type(bdec, jnp.float32)
    relu_e = jnp.maximum(e_s, 0.0)

    col = gt
    riota = jax.lax.broadcasted_iota(jnp.int32, (R, C), 0)
    d = 1
    while d < R:
        shifted = pltpu.roll(col, d, axis=1)
        col = col + jnp.where(riota >= d, shifted, 0.0)
        d *= 2
    colsum = jax.lax.slice(col, (0, R - 1, 0), (IB, R, C))
    liota = jax.lax.broadcasted_iota(jnp.int32, (1, C), 1)
    lane = colsum
    d = 1
    while d < C:
        shifted = pltpu.roll(lane, d, axis=2)
        lane = lane + jnp.where(liota >= d, shifted, 0.0)
        d *= 2
    G = jax.lax.slice(lane, (0, 0, C - 1), (IB, 1, C))
    lane_ex = lane - colsum
    c = col + lane_ex

    kpos = idx.astype(jnp.float32)
    U = G + (kpos + 1.0) - c
    Um1 = jnp.maximum(U - 1.0 + gt, 1.0)
    grad = gt / U + (1.0 - gt) * (G - c) / (U * Um1)
    loss = jnp.sum(relu_e * grad, axis=(1, 2), keepdims=True)
    loss = jnp.where(G > 0.0, loss, jnp.maximum(maxe, 0.0))
    out_ref[...] = jnp.broadcast_to(loss, (IB, 1, C)).astype(jnp.float32)


@jax.jit
def kernel(logits, labels):
    lg = logits.reshape(B, R, C)
    lb = labels.reshape(B, R, C)
    out = pl.pallas_call(
        _loss_kernel,
        grid=(B // IB,),
        in_specs=[
            pl.BlockSpec((IB, R, C), lambda i: (i, 0, 0)),
            pl.BlockSpec((IB, R, C), lambda i: (i, 0, 0)),
        ],
        out_specs=pl.BlockSpec((IB, 1, C), lambda i: (i, 0, 0)),
        out_shape=jax.ShapeDtypeStruct((B, 1, C), jnp.float32),
    )(lg, lb)
    return jnp.mean(out[:, 0, 0])


# speedup vs baseline: 1.1991x; 1.0466x over previous
import jax
import jax.numpy as jnp
from jax.experimental import pallas as pl
from jax.experimental.pallas import tpu as pltpu

R = 2048
C = 128
N = R * C
LOGN = 18
B = 8
IB = 4


def _loss_kernel(logits_ref, labels_ref, out_ref):
    lg = logits_ref[...]
    lb = labels_ref[...]
    lbf = lb.astype(jnp.float32)
    signs = 2.0 * lbf - 1.0
    e = 1.0 - lg * signs

    maxe = jnp.max(e, axis=(1, 2), keepdims=True)

    bits = jax.lax.bitcast_convert_type(e, jnp.int32)
    ikey = bits ^ ((bits >> 31) & jnp.int32(0x7FFFFFFF))
    x = (ikey & jnp.int32(~1)) | lb

    idx = (jax.lax.broadcasted_iota(jnp.int32, (R, C), 1) * R
           + jax.lax.broadcasted_iota(jnp.int32, (R, C), 0))

    gray = idx ^ (idx >> 1)
    w = x ^ (-((idx >> 1) & 1))
    for kk in range(1, LOGN + 1):
        for j in range(kk - 1, -1, -1):
            s = 1 << j
            if s >= R:
                t = s // R
                liota = jax.lax.broadcasted_iota(jnp.int32, (1, C), 1)
                is_lower = (liota & t) == 0
                down = pltpu.roll(w, C - t, axis=2)
                hi = jnp.maximum(w, down)
                lo = jnp.minimum(w, down)
                w = jnp.where(is_lower, hi, pltpu.roll(lo, t, axis=2))
            elif s >= 8:
                O = R // (2 * s)
                y = w.reshape(IB, O, 2, s, C)
                a = y[:, :, 0, :, :]
                b = y[:, :, 1, :, :]
                w = jnp.concatenate(
                    [jnp.maximum(a, b)[:, :, None], jnp.minimum(a, b)[:, :, None]],
                    axis=2).reshape(IB, R, C)
            else:
                y = w.reshape(IB, R // 8, 8, C)
                io8 = jax.lax.broadcasted_iota(jnp.int32, (1, 8, C), 1)
                is_lower = (io8 & s) == 0
                down = pltpu.roll(y, 8 - s, axis=2)
                hi = jnp.maximum(y, down)
                lo = jnp.minimum(y, down)
                w = jnp.where(is_lower, hi,
                              pltpu.roll(lo, s, axis=2)).reshape(IB, R, C)
        if kk < LOGN:
            w = w ^ ((gray << (31 - kk)) >> 31)
    x = w

    gt = (x & 1).astype(jnp.float32)
    bdec = jnp.where(x >= 0, x, x ^ jnp.int32(0x7FFFFFFF))
    e_s = jax.lax.bitcast_convert_type(bdec, jnp.float32)
    relu_e = jnp.maximum(e_s, 0.0)

    col = gt
    riota = jax.lax.broadcasted_iota(jnp.int32, (R, C), 0)
    d = 1
    while d < R:
        shifted = pltpu.roll(col, d, axis=1)
        col = col + jnp.where(riota >= d, shifted, 0.0)
        d *= 2
    colsum = jax.lax.slice(col, (0, R - 1, 0), (IB, R, C))
    liota = jax.lax.broadcasted_iota(jnp.int32, (1, C), 1)
    lane = colsum
    d = 1
    while d < C:
        shifted = pltpu.roll(lane, d, axis=2)
        lane = lane + jnp.where(liota >= d, shifted, 0.0)
        d *= 2
    G = jax.lax.slice(lane, (0, 0, C - 1), (IB, 1, C))
    lane_ex = lane - colsum
    c = col + lane_ex

    kpos = idx.astype(jnp.float32)
    U = G + (kpos + 1.0) - c
    Um1 = jnp.maximum(U - 1.0 + gt, 1.0)
    grad = gt / U + (1.0 - gt) * (G - c) / (U * Um1)
    loss = jnp.sum(relu_e * grad, axis=(1, 2), keepdims=True)
    loss = jnp.where(G > 0.0, loss, jnp.maximum(maxe, 0.0))
    out_ref[...] = jnp.broadcast_to(loss, (IB, 1, C)).astype(jnp.float32)


@jax.jit
def kernel(logits, labels):
    lg = logits.reshape(B, R, C)
    lb = labels.reshape(B, R, C)
    out = pl.pallas_call(
        _loss_kernel,
        grid=(B // IB,),
        in_specs=[
            pl.BlockSpec((IB, R, C), lambda i: (i, 0, 0)),
            pl.BlockSpec((IB, R, C), lambda i: (i, 0, 0)),
        ],
        out_specs=pl.BlockSpec((IB, 1, C), lambda i: (i, 0, 0)),
        out_shape=jax.ShapeDtypeStruct((B, 1, C), jnp.float32),
    )(lg, lb)
    return jnp.mean(out[:, 0, 0])
